# baseline (speedup 1.0000x reference)
import jax
import jax.numpy as jnp
from jax.experimental import pallas as pl
from jax.experimental.pallas import tpu as pltpu
from jax.experimental.pallas import tpu_sc as plsc

_ROWS = 16
_LANES = 16


def kernel(x, pos_table):
    B, T, D = x.shape
    x2 = x.reshape(B * T, D)
    nb_per_batch = T // _ROWS
    mesh = plsc.VectorSubcoreMesh(core_axis_name="c", subcore_axis_name="s")

    @pl.kernel(out_type=jax.ShapeDtypeStruct((B * T, D), x.dtype), mesh=mesh)
    def sc_kernel(x_hbm, pos_hbm, o_hbm):
        def body(x_vmem, pos_vmem, o_vmem):
            @pl.loop(0, _ROWS)
            def _(r):
                @pl.loop(0, D, step=_LANES)
                def _(c):
                    slc = (pl.ds(r, 1), pl.ds(c, _LANES))
                    o_vmem.at[*slc][...] = (
                        x_vmem.at[*slc][...] + pos_vmem.at[*slc][...]
                    )

        pltpu.emit_pipeline(
            body,
            grid=(B * T // _ROWS,),
            in_specs=[
                pl.BlockSpec((_ROWS, D), lambda i: (i, 0)),
                pl.BlockSpec((_ROWS, D), lambda i: (i % nb_per_batch, 0)),
            ],
            out_specs=[pl.BlockSpec((_ROWS, D), lambda i: (i, 0))],
            core_axis_name=("c", "s"),
            dimension_semantics=(pltpu.PARALLEL,),
        )(x_hbm, pos_hbm, o_hbm)

    return sc_kernel(x2, pos_table).reshape(B, T, D)


# speedup vs baseline: 4.3139x; 4.3139x over previous
import jax
import jax.numpy as jnp
from jax.experimental import pallas as pl

_BT = 512


def _add_block(x_ref, pos_ref, o_ref):
    o_ref[...] = x_ref[...] + pos_ref[...]


def kernel(x, pos_table):
    B, T, D = x.shape
    grid = (T // _BT,)
    return pl.pallas_call(
        _add_block,
        grid=grid,
        in_specs=[
            pl.BlockSpec((B, _BT, D), lambda t: (0, t, 0)),
            pl.BlockSpec((_BT, D), lambda t: (t, 0)),
        ],
        out_specs=pl.BlockSpec((B, _BT, D), lambda t: (0, t, 0)),
        out_shape=jax.ShapeDtypeStruct(x.shape, x.dtype),
    )(x, pos_table)


# speedup vs baseline: 4.3490x; 1.0081x over previous
import jax
import jax.numpy as jnp
from jax.experimental import pallas as pl

_BT = 2048


def _add_block(x_ref, pos_ref, o_ref):
    o_ref[...] = x_ref[...] + pos_ref[...]


def kernel(x, pos_table):
    B, T, D = x.shape
    grid = (T // _BT, B)
    return pl.pallas_call(
        _add_block,
        grid=grid,
        in_specs=[
            pl.BlockSpec((1, _BT, D), lambda t, b: (b, t, 0)),
            pl.BlockSpec((_BT, D), lambda t, b: (t, 0)),
        ],
        out_specs=pl.BlockSpec((1, _BT, D), lambda t, b: (b, t, 0)),
        out_shape=jax.ShapeDtypeStruct(x.shape, x.dtype),
    )(x, pos_table)
